# baseline (speedup 1.0000x reference)
import jax
import jax.numpy as jnp
from jax import lax
from jax.experimental import pallas as pl
from jax.experimental.pallas import tpu as pltpu
from jax.experimental.pallas import tpu_sc as plsc

N_NODES = 10000
N_EDGES = 160000
D = 256
DQ = 64
NS = 16
CH = 80
EPT = N_EDGES // NS
NCHUNK = EPT // CH
NR = 4
NI = 8
DLOOK = NR - 1
ILOOK = NI - 2
ROWS_PT = 624
TAIL_R0 = NS * ROWS_PT
TAIL_ROWS = N_NODES - TAIL_R0


def _gine_body(fq, gidx, eidx, dst, efq, out,
               sidx, eidxv, didx, grows, erows, tbl, acc, *sems):
    isem = sems[:NI]
    gsem = sems[NI:NI + NR]
    ssem = sems[NI + NR:]
    c = lax.axis_index("c")
    s = lax.axis_index("s")
    r0 = s * ROWS_PT
    ebase = s * EPT

    def run_pass(p):
        q = 2 * c + p

        pltpu.sync_copy(fq.at[pl.ds(q * N_NODES + r0, ROWS_PT)],
                        tbl.at[pl.ds(r0, ROWS_PT)])
        pltpu.sync_copy(fq.at[pl.ds(q * N_NODES + r0, ROWS_PT)],
                        acc.at[pl.ds(r0, ROWS_PT)])

        @pl.when(s == NS - 1)
        def _init_tail():
            pltpu.sync_copy(fq.at[pl.ds(q * N_NODES + TAIL_R0, TAIL_ROWS)],
                            tbl.at[pl.ds(TAIL_R0, TAIL_ROWS)])
            pltpu.sync_copy(fq.at[pl.ds(q * N_NODES + TAIL_R0, TAIL_ROWS)],
                            acc.at[pl.ds(TAIL_R0, TAIL_ROWS)])

        plsc.subcore_barrier()

        def prep_idx(j, ri):
            base = ebase + j * CH
            pltpu.async_copy(gidx.at[pl.ds(q * N_EDGES + base, CH)],
                             sidx.at[ri], isem[ri])
            pltpu.async_copy(eidx.at[pl.ds(q * N_EDGES + base, CH)],
                             eidxv.at[ri], isem[ri])
            pltpu.async_copy(dst.at[pl.ds(base, CH)], didx.at[ri], isem[ri])

        def prep_data(j, r, ri, first):
            if not first:
                pltpu.make_async_copy(fq.at[pl.ds(0, CH)], grows.at[r],
                                      ssem[r]).wait()
            for _ in range(3):
                pltpu.make_async_copy(gidx.at[pl.ds(0, CH)], sidx.at[ri],
                                      isem[ri]).wait()
            pltpu.async_copy(fq.at[sidx.at[ri]], grows.at[r], gsem[r])
            pltpu.async_copy(efq.at[eidxv.at[ri]], erows.at[r], gsem[r])

        def proc(r, ri):
            pltpu.make_async_copy(fq.at[pl.ds(0, CH)], grows.at[r],
                                  gsem[r]).wait()
            pltpu.make_async_copy(fq.at[pl.ds(0, CH)], erows.at[r],
                                  gsem[r]).wait()

            def edge(e, c2):
                e2 = e * 2
                for k in range(2):
                    for f in range(DQ // 16):
                        sl = pl.ds(f * 16, 16)
                        grows[r, e2 + k, sl] = jnp.maximum(
                            grows[r, e2 + k, sl] + erows[r, e2 + k, sl], 0.0)
                return c2

            lax.fori_loop(0, CH // 2, edge, 0)
            pltpu.async_copy(grows.at[r], acc.at[didx.at[ri]], ssem[r],
                             add=True)

        def substep(j, u, first_data, do_data, do_idx):
            if do_data:
                prep_data(j + DLOOK, (u + DLOOK) % NR, (u + DLOOK) % NI,
                          first_data)
            if do_idx:
                prep_idx(j + ILOOK, (u + ILOOK) % NI)
            proc(u % NR, u % NI)

        for j in range(ILOOK):
            prep_idx(j, j % NI)
        for j in range(DLOOK):
            prep_data(j, j % NR, j % NI, first=True)
        for j in range(NI):
            substep(j, j, first_data=(j + DLOOK < NR),
                    do_data=True, do_idx=True)

        def steady(t, carry):
            j0 = t * NI
            for u in range(NI):
                substep(j0 + u, u, first_data=False, do_data=True,
                        do_idx=True)
            return carry

        nsteady = (NCHUNK - ILOOK) // NI
        lax.fori_loop(1, nsteady, steady, 0)

        for j in range(nsteady * NI, NCHUNK):
            substep(j, j % NI, first_data=False,
                    do_data=j + DLOOK < NCHUNK, do_idx=j + ILOOK < NCHUNK)
        for j in range(NCHUNK - NR, NCHUNK):
            pltpu.make_async_copy(fq.at[pl.ds(0, CH)], grows.at[j % NR],
                                  ssem[j % NR]).wait()

        plsc.subcore_barrier()
        pltpu.sync_copy(acc.at[pl.ds(r0, ROWS_PT)],
                        out.at[q, pl.ds(r0, ROWS_PT)])

        @pl.when(s == NS - 1)
        def _out_tail():
            pltpu.sync_copy(acc.at[pl.ds(TAIL_R0, TAIL_ROWS)],
                            out.at[q, pl.ds(TAIL_R0, TAIL_ROWS)])

        plsc.subcore_barrier()

    run_pass(0)
    run_pass(1)


def kernel(feat, edge_index, efeat):
    src = edge_index[0].astype(jnp.int32)
    dst = edge_index[1].astype(jnp.int32)
    gidx = jnp.concatenate([src + q * N_NODES for q in range(4)])
    er = jnp.arange(N_EDGES, dtype=jnp.int32) * 4
    eidx = jnp.concatenate([er + q for q in range(4)])
    fq = jnp.concatenate([feat[:, q * DQ:(q + 1) * DQ] for q in range(4)])
    efq = efeat.reshape(4 * N_EDGES, DQ)
    mesh = plsc.VectorSubcoreMesh(core_axis_name="c", subcore_axis_name="s")
    k = pl.kernel(
        _gine_body,
        mesh=mesh,
        compiler_params=pltpu.CompilerParams(use_tc_tiling_on_sc=False),
        out_type=jax.ShapeDtypeStruct((4, N_NODES, DQ), jnp.float32),
        scratch_types=[
            pltpu.VMEM((NI, CH), jnp.int32),
            pltpu.VMEM((NI, CH), jnp.int32),
            pltpu.VMEM((NI, CH), jnp.int32),
            pltpu.VMEM((NR, CH, DQ), jnp.float32),
            pltpu.VMEM((NR, CH, DQ), jnp.float32),
            pltpu.VMEM_SHARED((N_NODES, DQ), jnp.float32),
            pltpu.VMEM_SHARED((N_NODES, DQ), jnp.float32),
        ] + [pltpu.SemaphoreType.DMA] * (NI + NR + NR),
    )
    out_q = k(fq, gidx, eidx, dst, efq)
    return out_q.transpose(1, 0, 2).reshape(N_NODES, D)
